# baseline (speedup 1.0000x reference)
import functools

import jax
import jax.numpy as jnp
from jax import lax
from jax.experimental import pallas as pl
from jax.experimental.pallas import tpu as pltpu
from jax.experimental.pallas import tpu_sc as plsc

F32 = jnp.float32
BF16 = jnp.bfloat16

NC = 2
NS = 16

_TC_PARAMS = pltpu.CompilerParams(dimension_semantics=("parallel",))


def _sc_gather_pair(tab_a, tab_b, ei, row_a, row_b, chunk, tc_tiling=True,
                    ch0=0, nch=None):
    n, d = tab_a.shape
    e = ei.shape[1]
    assert e % chunk == 0 and chunk % 8 == 0 and chunk <= 128
    if nch is None:
        nch = e // chunk
    eo = nch * chunk
    mesh = plsc.VectorSubcoreMesh(core_axis_name="c", subcore_axis_name="s")

    @functools.partial(
        pl.kernel,
        mesh=mesh,
        compiler_params=pltpu.CompilerParams(use_tc_tiling_on_sc=tc_tiling),
        out_type=(jax.ShapeDtypeStruct((eo, d), F32),
                  jax.ShapeDtypeStruct((eo, d), F32)),
        scratch_types=[
            pltpu.SemaphoreType.DMA,
            pltpu.SemaphoreType.DMA,
        ],
    )
    def k(ta_h, tb_h, ei_h, ga_h, gb_h, s1, s2):
        def body(ia_v, ib_v, ga_v, gb_v):
            cp_a = pltpu.async_copy(ta_h.at[ia_v.at[0]], ga_v, s1)
            cp_b = pltpu.async_copy(tb_h.at[ib_v.at[0]], gb_v, s2)
            cp_a.wait()
            cp_b.wait()

        pltpu.emit_pipeline(
            body,
            grid=(nch,),
            in_specs=[pl.BlockSpec((1, chunk), lambda i: (row_a, i + ch0)),
                      pl.BlockSpec((1, chunk), lambda i: (row_b, i + ch0))],
            out_specs=[pl.BlockSpec((chunk, d), lambda i: (i, 0)),
                       pl.BlockSpec((chunk, d), lambda i: (i, 0))],
            core_axis_name=("c", "s"),
            dimension_semantics=(pltpu.PARALLEL,),
        )(ei_h, ei_h, ga_h, gb_h)

    return k(tab_a, tab_b, ei)


def _sc_scatter_add(m, ei, dst_row, zeros, chunk, ch0_ei=0):
    e, d = m.shape
    n = zeros.shape[0]
    assert e % chunk == 0 and chunk % 8 == 0 and n % 8 == 0
    nch = e // chunk
    ch0 = ch0_ei
    octets = n // 8
    base_oct, extra = divmod(octets, NS)
    lo_rows = 8 * base_oct
    hi_rows = lo_rows + 8
    mesh = plsc.VectorSubcoreMesh(core_axis_name="c", subcore_axis_name="s")

    @functools.partial(
        pl.kernel,
        mesh=mesh,
        out_type=jax.ShapeDtypeStruct((NC, n, d), F32),
        scratch_types=[
            pltpu.VMEM_SHARED((n, d), F32),
        ],
    )
    def k(m_h, ei_h, z_h, out_h, acc_sh):
        cid = lax.axis_index("c")
        sid = lax.axis_index("s")
        r_hi = sid * hi_rows
        r_lo = extra * hi_rows + (sid - extra) * lo_rows

        @pl.when(sid < extra)
        def _():
            pltpu.sync_copy(z_h.at[pl.ds(r_hi, hi_rows)],
                            acc_sh.at[pl.ds(r_hi, hi_rows)])

        @pl.when(sid >= extra)
        def _():
            pltpu.sync_copy(z_h.at[pl.ds(r_lo, lo_rows)],
                            acc_sh.at[pl.ds(r_lo, lo_rows)])

        plsc.subcore_barrier()

        def body(idx_v, rows_v):
            pltpu.sync_copy(rows_v, acc_sh.at[idx_v.at[0]], add=True)

        pltpu.emit_pipeline(
            body,
            grid=(nch,),
            in_specs=[pl.BlockSpec((1, chunk), lambda i: (dst_row, i + ch0)),
                      pl.BlockSpec((chunk, d), lambda i: (i, 0))],
            out_specs=[],
            core_axis_name=("c", "s"),
            dimension_semantics=(pltpu.PARALLEL,),
        )(ei_h, m_h)

        plsc.subcore_barrier()

        @pl.when(sid < extra)
        def _():
            pltpu.sync_copy(acc_sh.at[pl.ds(r_hi, hi_rows)],
                            out_h.at[cid].at[pl.ds(r_hi, hi_rows)])

        @pl.when(sid >= extra)
        def _():
            pltpu.sync_copy(acc_sh.at[pl.ds(r_lo, lo_rows)],
                            out_h.at[cid].at[pl.ds(r_lo, lo_rows)])

    return k(m, ei, zeros)


def _mm(a, w):
    return lax.dot_general(a.astype(BF16), w, (((1,), (0,)), ((), ())),
                           preferred_element_type=F32)


def _ln(h, g, b, eps=1e-5):
    mu = jnp.mean(h, axis=-1, keepdims=True)
    var = jnp.mean((h - mu) * (h - mu), axis=-1, keepdims=True)
    return (h - mu) * lax.rsqrt(var + eps) * g + b


def _silu(x):
    return x * jax.nn.sigmoid(x)


def _row(p):
    return p.reshape(1, -1)


def _full(shape):
    return pl.BlockSpec(shape, lambda i: tuple(0 for _ in shape))


def _tc_encoder(x, w1, b1, g1, bn1, w2, b2, blk):
    n = x.shape[0]
    h_dim = w2.shape[1]

    def body(x_ref, w1_ref, b1_ref, g1_ref, bn1_ref, w2_ref, b2_ref, h_ref):
        t = _mm(x_ref[...], w1_ref[...]) + b1_ref[...]
        t = _silu(_ln(t, g1_ref[...], bn1_ref[...]))
        h_ref[...] = _mm(t, w2_ref[...]) + b2_ref[...]

    return pl.pallas_call(
        body,
        grid=(n // blk,),
        in_specs=[pl.BlockSpec((blk, 9), lambda i: (i, 0)),
                  _full(w1.shape), _full(b1.shape), _full(g1.shape),
                  _full(bn1.shape), _full(w2.shape), _full(b2.shape)],
        out_specs=pl.BlockSpec((blk, h_dim), lambda i: (i, 0)),
        out_shape=jax.ShapeDtypeStruct((n, h_dim), F32),
        compiler_params=_TC_PARAMS,
    )(x, w1, b1, g1, bn1, w2, b2)


def _tc_edge_attr(pa, pb, blk):
    e = pa.shape[0]

    def body(pa_ref, pb_ref, o_ref):
        d = pa_ref[:, 0:3] - pb_ref[:, 0:3]
        dist = jnp.sqrt(jnp.sum(d * d, axis=-1, keepdims=True)) + 1e-8
        o_ref[...] = jnp.concatenate([dist, d / dist], axis=-1)

    return pl.pallas_call(
        body,
        grid=(e // blk,),
        in_specs=[pl.BlockSpec((blk, 16), lambda i: (i, 0)),
                  pl.BlockSpec((blk, 16), lambda i: (i, 0))],
        out_specs=pl.BlockSpec((blk, 4), lambda i: (i, 0)),
        out_shape=jax.ShapeDtypeStruct((e, 4), F32),
        compiler_params=_TC_PARAMS,
    )(pa, pb)


def _tc_proj(h, wa, wb, blk):
    n, d = h.shape

    def body(h_ref, wa_ref, wb_ref, a_ref, b_ref):
        hb = h_ref[...].astype(BF16)
        dn = (((1,), (0,)), ((), ()))
        a_ref[...] = lax.dot_general(hb, wa_ref[...], dn,
                                     preferred_element_type=F32)
        b_ref[...] = lax.dot_general(hb, wb_ref[...], dn,
                                     preferred_element_type=F32)

    return pl.pallas_call(
        body,
        grid=(n // blk,),
        in_specs=[pl.BlockSpec((blk, d), lambda i: (i, 0)),
                  _full(wa.shape), _full(wb.shape)],
        out_specs=[pl.BlockSpec((blk, d), lambda i: (i, 0)),
                   pl.BlockSpec((blk, d), lambda i: (i, 0))],
        out_shape=[jax.ShapeDtypeStruct((n, d), F32),
                   jax.ShapeDtypeStruct((n, d), F32)],
        compiler_params=_TC_PARAMS,
    )(h, wa, wb)


def _edge_mlp_tail(t, g1, bn1, w2, b2, g2, bn2, w3, b3):
    t = _silu(_ln(t, g1, bn1))
    t = _mm(t, w2) + b2
    t = _silu(_ln(t, g2, bn2))
    return _mm(t, w3) + b3


def _tc_edge_mlp(ga, gb, ea, w1c, b1, g1, bn1, w2, b2, g2, bn2, w3, b3, blk,
                 ea_blk0=0):
    e, d = ga.shape

    def body(ga_ref, gb_ref, ea_ref, w1c_ref, b1_ref, g1_ref, bn1_ref,
             w2_ref, b2_ref, g2_ref, bn2_ref, w3_ref, b3_ref, m_ref):
        t = (ga_ref[...] + gb_ref[...]
             + _mm(ea_ref[...], w1c_ref[...]) + b1_ref[...])
        m_ref[...] = _edge_mlp_tail(t, g1_ref[...], bn1_ref[...], w2_ref[...],
                                    b2_ref[...], g2_ref[...], bn2_ref[...],
                                    w3_ref[...], b3_ref[...])

    return pl.pallas_call(
        body,
        grid=(e // blk,),
        in_specs=[pl.BlockSpec((blk, d), lambda i: (i, 0)),
                  pl.BlockSpec((blk, d), lambda i: (i, 0)),
                  pl.BlockSpec((blk, 4), lambda i: (i + ea_blk0, 0)),
                  _full(w1c.shape), _full(b1.shape), _full(g1.shape),
                  _full(bn1.shape), _full(w2.shape), _full(b2.shape),
                  _full(g2.shape), _full(bn2.shape), _full(w3.shape),
                  _full(b3.shape)],
        out_specs=pl.BlockSpec((blk, d), lambda i: (i, 0)),
        out_shape=jax.ShapeDtypeStruct((e, d), F32),
        compiler_params=_TC_PARAMS,
    )(ga, gb, ea, w1c, b1, g1, bn1, w2, b2, g2, bn2, w3, b3)


def _tc_node_mlp(h, parts, w1h, w1a, b1, g1, bn1, w2, b2, gn, bn, blk):
    n, d = h.shape
    k = len(parts)

    def body(*refs):
        h_ref = refs[0]
        p_refs = refs[1:1 + k]
        (w1h_ref, w1a_ref, b1_ref, g1_ref, bn1_ref, w2_ref, b2_ref, gn_ref,
         bn_ref, o_ref) = refs[1 + k:]
        hv = h_ref[...]
        aggr = p_refs[0][...]
        for pr in p_refs[1:]:
            aggr = aggr + pr[...]
        u = _mm(hv, w1h_ref[...]) + _mm(aggr, w1a_ref[...]) + b1_ref[...]
        u = _silu(_ln(u, g1_ref[...], bn1_ref[...]))
        hn = _mm(u, w2_ref[...]) + b2_ref[...]
        o_ref[...] = _ln(hv + hn, gn_ref[...], bn_ref[...])

    row_spec = pl.BlockSpec((blk, d), lambda i: (i, 0))
    return pl.pallas_call(
        body,
        grid=(n // blk,),
        in_specs=[row_spec] * (1 + k) + [
            _full(w1h.shape), _full(w1a.shape), _full(b1.shape),
            _full(g1.shape), _full(bn1.shape), _full(w2.shape),
            _full(b2.shape), _full(gn.shape), _full(bn.shape)],
        out_specs=row_spec,
        out_shape=jax.ShapeDtypeStruct((n, d), F32),
        compiler_params=_TC_PARAMS,
    )(h, *parts, w1h, w1a, b1, g1, bn1, w2, b2, gn, bn)


def _tc_decoder(h, w1, b1, w2, b2, w3, b3, blk):
    n, d = h.shape

    def body(h_ref, w1_ref, b1_ref, w2_ref, b2_ref, w3_ref, b3_ref, o_ref):
        t = _silu(_mm(h_ref[...], w1_ref[...]) + b1_ref[...])
        t = _silu(_mm(t, w2_ref[...]) + b2_ref[...])
        o_ref[...] = _mm(t, w3_ref[...]) + b3_ref[...]

    return pl.pallas_call(
        body,
        grid=(n // blk,),
        in_specs=[pl.BlockSpec((blk, d), lambda i: (i, 0)),
                  _full(w1.shape), _full(b1.shape), _full(w2.shape),
                  _full(b2.shape), _full(w3.shape), _full(b3.shape)],
        out_specs=pl.BlockSpec((blk, 6), lambda i: (i, 0)),
        out_shape=jax.ShapeDtypeStruct((n, 6), F32),
        compiler_params=_TC_PARAMS,
    )(h, w1, b1, w2, b2, w3, b3)


def kernel(x, edge_index, pos, params):
    n = x.shape[0]
    e = edge_index.shape[1]
    h_dim = params["enc"]["l2"]["w"].shape[1]

    ei = edge_index.astype(jnp.int32)
    SRC, DST = 0, 1

    def pick_blk(size):
        for cand in (2000, 1000, 800, 400, 200, 80, 8):
            if size % cand == 0:
                return cand
        return size

    blk_n = pick_blk(n)
    blk_e = pick_blk(e)
    chunk = 128 if e % 128 == 0 else 80

    def w(p):
        return p["w"].astype(BF16)

    def b(p):
        return _row(p["b"])

    posp = jnp.concatenate([pos, jnp.zeros((n, 13), F32)], axis=1)
    pa, pb = _sc_gather_pair(posp, posp, ei, DST, SRC, chunk, tc_tiling=False)
    ea = _tc_edge_attr(pa, pb, blk_e)

    enc = params["enc"]
    h = _tc_encoder(x, w(enc["l1"]), b(enc["l1"]), _row(enc["ln1"]["g"]),
                    _row(enc["ln1"]["b"]), w(enc["l2"]), b(enc["l2"]), blk_n)

    n_slices = 2 if e % (2 * chunk * blk_e) == 0 else 1
    es = e // n_slices
    nch_s = es // chunk

    zeros = jnp.zeros((n, h_dim), F32)
    for lp in params["layers"]:
        ep = lp["edge"]
        w1 = ep["l1"]["w"]
        a_tab, b_tab = _tc_proj(h, w1[:h_dim].astype(BF16),
                                w1[h_dim:2 * h_dim].astype(BF16), blk_n)
        gs = [_sc_gather_pair(a_tab, b_tab, ei, DST, SRC, chunk,
                              ch0=s * nch_s, nch=nch_s)
              for s in range(n_slices)]
        parts = []
        for s in range(n_slices):
            ga, gb = gs[s]
            m = _tc_edge_mlp(
                ga, gb, ea, w1[2 * h_dim:].astype(BF16), b(ep["l1"]),
                _row(ep["ln1"]["g"]), _row(ep["ln1"]["b"]), w(ep["l2"]),
                b(ep["l2"]), _row(ep["ln2"]["g"]), _row(ep["ln2"]["b"]),
                w(ep["l3"]), b(ep["l3"]), blk_e, ea_blk0=s * (es // blk_e))
            part = _sc_scatter_add(m, ei, DST, zeros, chunk,
                                   ch0_ei=s * nch_s)
            parts.extend([part[0], part[1]])
        np_ = lp["node"]
        wn1 = np_["l1"]["w"]
        h = _tc_node_mlp(
            h, parts, wn1[:h_dim].astype(BF16),
            wn1[h_dim:].astype(BF16), b(np_["l1"]), _row(np_["ln1"]["g"]),
            _row(np_["ln1"]["b"]), w(np_["l2"]), b(np_["l2"]),
            _row(lp["norm"]["g"]), _row(lp["norm"]["b"]), blk_n)

    dec = params["dec"]
    return _tc_decoder(h, w(dec["l1"]), b(dec["l1"]), w(dec["l2"]),
                       b(dec["l2"]), w(dec["l3"]), b(dec["l3"]), blk_n)


# speedup vs baseline: 1.2265x; 1.2265x over previous
import functools

import jax
import jax.numpy as jnp
from jax import lax
from jax.experimental import pallas as pl
from jax.experimental.pallas import tpu as pltpu
from jax.experimental.pallas import tpu_sc as plsc

F32 = jnp.float32
BF16 = jnp.bfloat16

NC = 2
NS = 16

_TC_PARAMS = pltpu.CompilerParams(dimension_semantics=("parallel",))


def _sc_gather_pair(tab_a, tab_b, ei, row_a, row_b, chunk, tc_tiling=True,
                    ch0=0, nch=None):
    n, d = tab_a.shape
    e = ei.shape[1]
    assert e % chunk == 0 and chunk % 8 == 0 and chunk <= 128
    if nch is None:
        nch = e // chunk
    eo = nch * chunk
    mesh = plsc.VectorSubcoreMesh(core_axis_name="c", subcore_axis_name="s")

    @functools.partial(
        pl.kernel,
        mesh=mesh,
        compiler_params=pltpu.CompilerParams(use_tc_tiling_on_sc=tc_tiling),
        out_type=(jax.ShapeDtypeStruct((eo, d), F32),
                  jax.ShapeDtypeStruct((eo, d), F32)),
        scratch_types=[
            pltpu.SemaphoreType.DMA,
            pltpu.SemaphoreType.DMA,
        ],
    )
    def k(ta_h, tb_h, ei_h, ga_h, gb_h, s1, s2):
        def body(ia_v, ib_v, ga_v, gb_v):
            cp_a = pltpu.async_copy(ta_h.at[ia_v.at[0]], ga_v, s1)
            cp_b = pltpu.async_copy(tb_h.at[ib_v.at[0]], gb_v, s2)
            cp_a.wait()
            cp_b.wait()

        pltpu.emit_pipeline(
            body,
            grid=(nch,),
            in_specs=[pl.BlockSpec((1, chunk), lambda i: (row_a, i + ch0)),
                      pl.BlockSpec((1, chunk), lambda i: (row_b, i + ch0))],
            out_specs=[pl.BlockSpec((chunk, d), lambda i: (i, 0)),
                       pl.BlockSpec((chunk, d), lambda i: (i, 0))],
            core_axis_name=("c", "s"),
            dimension_semantics=(pltpu.PARALLEL,),
        )(ei_h, ei_h, ga_h, gb_h)

    return k(tab_a, tab_b, ei)


def _sc_scatter_add(m, ei, dst_row, zeros, chunk, ch0_ei=0):
    e, d = m.shape
    n = zeros.shape[0]
    assert e % chunk == 0 and chunk % 8 == 0 and n % 8 == 0
    nch = e // chunk
    ch0 = ch0_ei
    octets = n // 8
    base_oct, extra = divmod(octets, NS)
    lo_rows = 8 * base_oct
    hi_rows = lo_rows + 8
    mesh = plsc.VectorSubcoreMesh(core_axis_name="c", subcore_axis_name="s")

    @functools.partial(
        pl.kernel,
        mesh=mesh,
        out_type=jax.ShapeDtypeStruct((NC, n, d), F32),
        scratch_types=[
            pltpu.VMEM_SHARED((n, d), F32),
        ],
    )
    def k(m_h, ei_h, z_h, out_h, acc_sh):
        cid = lax.axis_index("c")
        sid = lax.axis_index("s")
        r_hi = sid * hi_rows
        r_lo = extra * hi_rows + (sid - extra) * lo_rows

        @pl.when(sid < extra)
        def _():
            pltpu.sync_copy(z_h.at[pl.ds(r_hi, hi_rows)],
                            acc_sh.at[pl.ds(r_hi, hi_rows)])

        @pl.when(sid >= extra)
        def _():
            pltpu.sync_copy(z_h.at[pl.ds(r_lo, lo_rows)],
                            acc_sh.at[pl.ds(r_lo, lo_rows)])

        plsc.subcore_barrier()

        def body(idx_v, rows_v):
            pltpu.sync_copy(rows_v, acc_sh.at[idx_v.at[0]], add=True)

        pltpu.emit_pipeline(
            body,
            grid=(nch,),
            in_specs=[pl.BlockSpec((1, chunk), lambda i: (dst_row, i + ch0)),
                      pl.BlockSpec((chunk, d), lambda i: (i, 0))],
            out_specs=[],
            core_axis_name=("c", "s"),
            dimension_semantics=(pltpu.PARALLEL,),
        )(ei_h, m_h)

        plsc.subcore_barrier()

        @pl.when(sid < extra)
        def _():
            pltpu.sync_copy(acc_sh.at[pl.ds(r_hi, hi_rows)],
                            out_h.at[cid].at[pl.ds(r_hi, hi_rows)])

        @pl.when(sid >= extra)
        def _():
            pltpu.sync_copy(acc_sh.at[pl.ds(r_lo, lo_rows)],
                            out_h.at[cid].at[pl.ds(r_lo, lo_rows)])

    return k(m, ei, zeros)


def _mm(a, w):
    return lax.dot_general(a.astype(BF16), w, (((1,), (0,)), ((), ())),
                           preferred_element_type=F32)


def _ln(h, g, b, eps=1e-5):
    mu = jnp.mean(h, axis=-1, keepdims=True)
    var = jnp.mean((h - mu) * (h - mu), axis=-1, keepdims=True)
    return (h - mu) * lax.rsqrt(var + eps) * g + b


def _silu(x):
    return x * jax.nn.sigmoid(x)


def _row(p):
    return p.reshape(1, -1)


def _full(shape):
    return pl.BlockSpec(shape, lambda i: tuple(0 for _ in shape))


def _tc_encoder(x, w1, b1, g1, bn1, w2, b2, blk):
    n = x.shape[0]
    h_dim = w2.shape[1]

    def body(x_ref, w1_ref, b1_ref, g1_ref, bn1_ref, w2_ref, b2_ref, h_ref):
        t = _mm(x_ref[...], w1_ref[...]) + b1_ref[...]
        t = _silu(_ln(t, g1_ref[...], bn1_ref[...]))
        h_ref[...] = _mm(t, w2_ref[...]) + b2_ref[...]

    return pl.pallas_call(
        body,
        grid=(n // blk,),
        in_specs=[pl.BlockSpec((blk, 9), lambda i: (i, 0)),
                  _full(w1.shape), _full(b1.shape), _full(g1.shape),
                  _full(bn1.shape), _full(w2.shape), _full(b2.shape)],
        out_specs=pl.BlockSpec((blk, h_dim), lambda i: (i, 0)),
        out_shape=jax.ShapeDtypeStruct((n, h_dim), F32),
        compiler_params=_TC_PARAMS,
    )(x, w1, b1, g1, bn1, w2, b2)


def _tc_edge_attr(pa, pb, blk):
    e, d_in = pa.shape

    def body(pa_ref, pb_ref, o_ref):
        d = pa_ref[:, 0:3] - pb_ref[:, 0:3]
        dist = jnp.sqrt(jnp.sum(d * d, axis=-1, keepdims=True)) + 1e-8
        ea = jnp.concatenate([dist, d / dist], axis=-1)
        o_ref[...] = ea.T

    return pl.pallas_call(
        body,
        grid=(e // blk,),
        in_specs=[pl.BlockSpec((blk, d_in), lambda i: (i, 0)),
                  pl.BlockSpec((blk, d_in), lambda i: (i, 0))],
        out_specs=pl.BlockSpec((4, blk), lambda i: (0, i)),
        out_shape=jax.ShapeDtypeStruct((4, e), F32),
        compiler_params=_TC_PARAMS,
    )(pa, pb)


def _tc_proj(h, wa, wb, blk):
    n, d = h.shape

    def body(h_ref, wa_ref, wb_ref, a_ref, b_ref):
        hb = h_ref[...].astype(BF16)
        dn = (((1,), (0,)), ((), ()))
        a_ref[...] = lax.dot_general(hb, wa_ref[...], dn,
                                     preferred_element_type=F32)
        b_ref[...] = lax.dot_general(hb, wb_ref[...], dn,
                                     preferred_element_type=F32)

    return pl.pallas_call(
        body,
        grid=(n // blk,),
        in_specs=[pl.BlockSpec((blk, d), lambda i: (i, 0)),
                  _full(wa.shape), _full(wb.shape)],
        out_specs=[pl.BlockSpec((blk, d), lambda i: (i, 0)),
                   pl.BlockSpec((blk, d), lambda i: (i, 0))],
        out_shape=[jax.ShapeDtypeStruct((n, d), F32),
                   jax.ShapeDtypeStruct((n, d), F32)],
        compiler_params=_TC_PARAMS,
    )(h, wa, wb)


def _edge_mlp_tail(t, g1, bn1, w2, b2, g2, bn2, w3, b3):
    t = _silu(_ln(t, g1, bn1))
    t = _mm(t, w2) + b2
    t = _silu(_ln(t, g2, bn2))
    return _mm(t, w3) + b3


def _tc_edge_mlp(ga, gb, ea, w1c, b1, g1, bn1, w2, b2, g2, bn2, w3, b3, blk,
                 ea_blk0=0):
    e, d = ga.shape

    def body(ga_ref, gb_ref, ea_ref, w1c_ref, b1_ref, g1_ref, bn1_ref,
             w2_ref, b2_ref, g2_ref, bn2_ref, w3_ref, b3_ref, m_ref):
        ea_c = lax.dot_general(ea_ref[...].astype(BF16), w1c_ref[...],
                               (((0,), (0,)), ((), ())),
                               preferred_element_type=F32)
        t = ga_ref[...] + gb_ref[...] + ea_c + b1_ref[...]
        m_ref[...] = _edge_mlp_tail(t, g1_ref[...], bn1_ref[...], w2_ref[...],
                                    b2_ref[...], g2_ref[...], bn2_ref[...],
                                    w3_ref[...], b3_ref[...])

    return pl.pallas_call(
        body,
        grid=(e // blk,),
        in_specs=[pl.BlockSpec((blk, d), lambda i: (i, 0)),
                  pl.BlockSpec((blk, d), lambda i: (i, 0)),
                  pl.BlockSpec((4, blk), lambda i: (0, i + ea_blk0)),
                  _full(w1c.shape), _full(b1.shape), _full(g1.shape),
                  _full(bn1.shape), _full(w2.shape), _full(b2.shape),
                  _full(g2.shape), _full(bn2.shape), _full(w3.shape),
                  _full(b3.shape)],
        out_specs=pl.BlockSpec((blk, d), lambda i: (i, 0)),
        out_shape=jax.ShapeDtypeStruct((e, d), F32),
        compiler_params=_TC_PARAMS,
    )(ga, gb, ea, w1c, b1, g1, bn1, w2, b2, g2, bn2, w3, b3)


def _tc_node_mlp(h, parts, w1h, w1a, b1, g1, bn1, w2, b2, gn, bn, blk):
    n, d = h.shape
    k = len(parts)

    def body(*refs):
        h_ref = refs[0]
        p_refs = refs[1:1 + k]
        (w1h_ref, w1a_ref, b1_ref, g1_ref, bn1_ref, w2_ref, b2_ref, gn_ref,
         bn_ref, o_ref) = refs[1 + k:]
        hv = h_ref[...]
        aggr = p_refs[0][...]
        for pr in p_refs[1:]:
            aggr = aggr + pr[...]
        u = _mm(hv, w1h_ref[...]) + _mm(aggr, w1a_ref[...]) + b1_ref[...]
        u = _silu(_ln(u, g1_ref[...], bn1_ref[...]))
        hn = _mm(u, w2_ref[...]) + b2_ref[...]
        o_ref[...] = _ln(hv + hn, gn_ref[...], bn_ref[...])

    row_spec = pl.BlockSpec((blk, d), lambda i: (i, 0))
    return pl.pallas_call(
        body,
        grid=(n // blk,),
        in_specs=[row_spec] * (1 + k) + [
            _full(w1h.shape), _full(w1a.shape), _full(b1.shape),
            _full(g1.shape), _full(bn1.shape), _full(w2.shape),
            _full(b2.shape), _full(gn.shape), _full(bn.shape)],
        out_specs=row_spec,
        out_shape=jax.ShapeDtypeStruct((n, d), F32),
        compiler_params=_TC_PARAMS,
    )(h, *parts, w1h, w1a, b1, g1, bn1, w2, b2, gn, bn)


def _tc_decoder(h, w1, b1, w2, b2, w3, b3, blk):
    n, d = h.shape

    def body(h_ref, w1_ref, b1_ref, w2_ref, b2_ref, w3_ref, b3_ref, o_ref):
        t = _silu(_mm(h_ref[...], w1_ref[...]) + b1_ref[...])
        t = _silu(_mm(t, w2_ref[...]) + b2_ref[...])
        o_ref[...] = _mm(t, w3_ref[...]) + b3_ref[...]

    return pl.pallas_call(
        body,
        grid=(n // blk,),
        in_specs=[pl.BlockSpec((blk, d), lambda i: (i, 0)),
                  _full(w1.shape), _full(b1.shape), _full(w2.shape),
                  _full(b2.shape), _full(w3.shape), _full(b3.shape)],
        out_specs=pl.BlockSpec((blk, 6), lambda i: (i, 0)),
        out_shape=jax.ShapeDtypeStruct((n, 6), F32),
        compiler_params=_TC_PARAMS,
    )(h, w1, b1, w2, b2, w3, b3)


def kernel(x, edge_index, pos, params):
    n = x.shape[0]
    e = edge_index.shape[1]
    h_dim = params["enc"]["l2"]["w"].shape[1]

    ei = edge_index.astype(jnp.int32)
    SRC, DST = 0, 1

    def pick_blk(size):
        for cand in (2000, 1000, 800, 400, 200, 80, 8):
            if size % cand == 0:
                return cand
        return size

    blk_n = pick_blk(n)
    blk_e = next((c for c in (3200, 1280, 640, 128) if e % c == 0), e)
    chunk = 128 if e % 128 == 0 else 80

    def w(p):
        return p["w"].astype(BF16)

    def b(p):
        return _row(p["b"])

    posp = jnp.concatenate([pos, jnp.zeros((n, 125), F32)], axis=1)
    pa, pb = _sc_gather_pair(posp, posp, ei, DST, SRC, chunk)
    ea = _tc_edge_attr(pa, pb, blk_e)

    enc = params["enc"]
    h = _tc_encoder(x, w(enc["l1"]), b(enc["l1"]), _row(enc["ln1"]["g"]),
                    _row(enc["ln1"]["b"]), w(enc["l2"]), b(enc["l2"]), blk_n)

    n_slices = (2 if e % 2 == 0 and (e // 2) % chunk == 0
                and (e // 2) % blk_e == 0 else 1)
    es = e // n_slices
    nch_s = es // chunk

    zeros = jnp.zeros((n, h_dim), F32)
    for lp in params["layers"]:
        ep = lp["edge"]
        w1 = ep["l1"]["w"]
        a_tab, b_tab = _tc_proj(h, w1[:h_dim].astype(BF16),
                                w1[h_dim:2 * h_dim].astype(BF16), blk_n)
        gs = [_sc_gather_pair(a_tab, b_tab, ei, DST, SRC, chunk,
                              ch0=s * nch_s, nch=nch_s)
              for s in range(n_slices)]
        parts = []
        for s in range(n_slices):
            ga, gb = gs[s]
            m = _tc_edge_mlp(
                ga, gb, ea, w1[2 * h_dim:].astype(BF16), b(ep["l1"]),
                _row(ep["ln1"]["g"]), _row(ep["ln1"]["b"]), w(ep["l2"]),
                b(ep["l2"]), _row(ep["ln2"]["g"]), _row(ep["ln2"]["b"]),
                w(ep["l3"]), b(ep["l3"]), blk_e, ea_blk0=s * (es // blk_e))
            part = _sc_scatter_add(m, ei, DST, zeros, chunk,
                                   ch0_ei=s * nch_s)
            parts.extend([part[0], part[1]])
        np_ = lp["node"]
        wn1 = np_["l1"]["w"]
        h = _tc_node_mlp(
            h, parts, wn1[:h_dim].astype(BF16),
            wn1[h_dim:].astype(BF16), b(np_["l1"]), _row(np_["ln1"]["g"]),
            _row(np_["ln1"]["b"]), w(np_["l2"]), b(np_["l2"]),
            _row(lp["norm"]["g"]), _row(lp["norm"]["b"]), blk_n)

    dec = params["dec"]
    return _tc_decoder(h, w(dec["l1"]), b(dec["l1"]), w(dec["l2"]),
                       b(dec["l2"]), w(dec["l3"]), b(dec["l3"]), blk_n)
